# Initial kernel scaffold; baseline (speedup 1.0000x reference)
import jax
import jax.numpy as jnp
from jax.experimental import pallas as pl


def kernel(x, cas_rgb_w1, cas_rgb_b1, cas_rgb_w2, cas_rgb_b2, cas_flow_w1, cas_flow_b1, cas_flow_w2, cas_flow_b2, aas_rgb_w1, aas_rgb_b1, aas_rgb_w2, aas_rgb_b2, aas_flow_w1, aas_flow_b1, aas_flow_w2, aas_flow_b2):
    raise NotImplementedError("write your pallas kernel here")


# speedup vs baseline: 6.6703x; 6.6703x over previous
import jax
import jax.numpy as jnp
from jax import lax
from jax.experimental import pallas as pl

_N, _T, _C = 4, 512, 20
_R = _N * _C
_OC = 32


def _enc_body(x_ref, w_ref, b1_ref, w2_ref, b2_ref, out_ref):
    xp = x_ref[0, 0, 0:_T, :]
    xc = x_ref[0, 0, 1:_T + 1, :]
    xn = x_ref[0, 0, 2:_T + 2, :]
    h = jnp.dot(xp, w_ref[0, 0], preferred_element_type=jnp.float32)
    h = h + jnp.dot(xc, w_ref[0, 1], preferred_element_type=jnp.float32)
    h = h + jnp.dot(xn, w_ref[0, 2], preferred_element_type=jnp.float32)
    h = jnp.maximum(h + b1_ref[0], 0.0)
    out = jnp.dot(h, w2_ref[0], preferred_element_type=jnp.float32)
    out_ref[0, 0] = out + b2_ref[0]


def _run_encoders(xpad, w1s, b1s, w2s, b2s):
    return pl.pallas_call(
        _enc_body,
        grid=(4, _N),
        in_specs=[
            pl.BlockSpec((1, 1, _T + 2, 1024), lambda e, n: (n, e % 2, 0, 0)),
            pl.BlockSpec((1, 3, 1024, 1024), lambda e, n: (e, 0, 0, 0)),
            pl.BlockSpec((1, 1, 1024), lambda e, n: (e, 0, 0)),
            pl.BlockSpec((1, 1024, _OC), lambda e, n: (e, 0, 0)),
            pl.BlockSpec((1, 1, _OC), lambda e, n: (e, 0, 0)),
        ],
        out_specs=pl.BlockSpec((1, 1, _T, _OC), lambda e, n: (e, n, 0, 0)),
        out_shape=jax.ShapeDtypeStruct((4, _N, _T, _OC), jnp.float32),
    )(xpad, w1s, b1s, w2s, b2s)


def _combine_body(o_ref, seg_ref, cassum_ref, aas_ref):
    cas = o_ref[0, 0] + o_ref[1, 0]
    aassum = o_ref[2, 0] + o_ref[3, 0]
    lanes = lax.broadcasted_iota(jnp.int32, (_T, _OC), 1)
    valid = lanes < _C
    mx = jnp.max(jnp.where(valid, cas, -jnp.inf), axis=1, keepdims=True)
    e = jnp.where(valid, jnp.exp(cas - mx), 0.0)
    soft = e / jnp.sum(e, axis=1, keepdims=True)
    sig = 1.0 / (1.0 + jnp.exp(-aassum))
    seg = (soft + sig[:, 0:1]) * 0.5
    seg_ref[0] = jnp.where(valid, seg, 0.0)
    cassum_ref[0] = jnp.where(valid, cas, 0.0)
    aas_ref[0] = sig


def _run_combine(out_all):
    return pl.pallas_call(
        _combine_body,
        grid=(_N,),
        in_specs=[pl.BlockSpec((4, 1, _T, _OC), lambda n: (0, n, 0, 0))],
        out_specs=[
            pl.BlockSpec((1, _T, _OC), lambda n: (n, 0, 0)),
            pl.BlockSpec((1, _T, _OC), lambda n: (n, 0, 0)),
            pl.BlockSpec((1, _T, _OC), lambda n: (n, 0, 0)),
        ],
        out_shape=[
            jax.ShapeDtypeStruct((_N, _T, _OC), jnp.float32),
            jax.ShapeDtypeStruct((_N, _T, _OC), jnp.float32),
            jax.ShapeDtypeStruct((_N, _T, _OC), jnp.float32),
        ],
    )(out_all)


def _rank_body(row_ref, col_ref, sorted_ref, rank_ref):
    b = row_ref[0]
    a = col_ref[0]
    iu = lax.broadcasted_iota(jnp.int32, (_T, _T), 0)
    it = lax.broadcasted_iota(jnp.int32, (_T, _T), 1)
    beats = ((a > b) | ((a == b) & (iu < it))).astype(jnp.float32)
    rank_lane = jnp.sum(beats, axis=0, keepdims=True)
    rank_sub = float(_T - 1) - jnp.sum(beats, axis=1, keepdims=True)
    isub = lax.broadcasted_iota(jnp.int32, (_T, _T), 0).astype(jnp.float32)
    onehot = (rank_lane == isub).astype(jnp.float32)
    sorted_ref[0] = jnp.sum(onehot * b, axis=1, keepdims=True)
    rank_ref[0] = rank_sub


def _run_rank(seg_rows, seg_cols):
    return pl.pallas_call(
        _rank_body,
        grid=(_R,),
        in_specs=[
            pl.BlockSpec((1, 1, _T), lambda r: (r, 0, 0)),
            pl.BlockSpec((1, _T, 1), lambda r: (r, 0, 0)),
        ],
        out_specs=[
            pl.BlockSpec((1, _T, 1), lambda r: (r, 0, 0)),
            pl.BlockSpec((1, _T, 1), lambda r: (r, 0, 0)),
        ],
        out_shape=[
            jax.ShapeDtypeStruct((_R, _T, 1), jnp.float32),
            jax.ShapeDtypeStruct((_R, _T, 1), jnp.float32),
        ],
    )(seg_rows, seg_cols)


def _cluster_body(sv_ref, out_ref):
    v0 = sv_ref[0:1, :]
    vlast = sv_ref[_T - 1:_T, :]
    out_ref[0:1, :] = jnp.ones_like(v0)
    out_ref[_T - 1:_T, :] = jnp.zeros_like(v0)

    def body(i, carry):
        ps, pn, ns_, nn = carry
        vi = sv_ref[pl.ds(i, 1), :]
        cond = (jnp.abs(vi - ps / pn) <= jnp.abs(vi - ns_ / nn)).astype(
            jnp.float32)
        out_ref[pl.ds(i, 1), :] = cond
        fi = (i + 1).astype(jnp.float32)
        return (ps + cond * vi / fi, pn + cond / fi,
                ns_ + (1.0 - cond) * vi, nn + (1.0 - cond))

    lax.fori_loop(1, _T - 1, body,
                  (v0, jnp.ones_like(v0), vlast, jnp.ones_like(v0)))
    mask = out_ref[...]
    sv = sv_ref[...]
    cnt = jnp.sum(mask, axis=0, keepdims=True)
    cnt = jnp.where(cnt == 0.0, 1.0, cnt)
    act = jnp.sum(mask * sv, axis=0, keepdims=True) / cnt
    out_ref[...] = jnp.where(sv >= act, mask, 0.0)


def _run_cluster(sorted_cols):
    return pl.pallas_call(
        _cluster_body,
        out_shape=jax.ShapeDtypeStruct((_T, _R), jnp.float32),
    )(sorted_cols)


def _unsort_body(rank_ref, refined_ref, out_ref):
    r = rank_ref[0]
    q = refined_ref[0]
    ilane = lax.broadcasted_iota(jnp.int32, (_T, _T), 1).astype(jnp.float32)
    sel = (r == ilane).astype(jnp.float32)
    out_ref[0] = jnp.sum(sel * q, axis=1, keepdims=True)


def _run_unsort(rank_rows, refined_rows):
    return pl.pallas_call(
        _unsort_body,
        grid=(_R,),
        in_specs=[
            pl.BlockSpec((1, _T, 1), lambda r: (r, 0, 0)),
            pl.BlockSpec((1, 1, _T), lambda r: (r, 0, 0)),
        ],
        out_specs=pl.BlockSpec((1, _T, 1), lambda r: (r, 0, 0)),
        out_shape=jax.ShapeDtypeStruct((_R, _T, 1), jnp.float32),
    )(rank_rows, refined_rows)


def _masked_softmax(x, valid):
    mx = jnp.max(jnp.where(valid, x, -jnp.inf), axis=1, keepdims=True)
    e = jnp.where(valid, jnp.exp(x - mx), 0.0)
    return e / jnp.sum(e, axis=1, keepdims=True)


def _final_body(cas_ref, mask_ref, act_ref, bkg_ref):
    cas = cas_ref[0]
    m = mask_ref[0]
    valid = lax.broadcasted_iota(jnp.int32, (1, _OC), 1) < _C
    an = jnp.sum(m, axis=0, keepdims=True)
    an = jnp.where(an == 0.0, 1.0, an)
    bn = jnp.sum(1.0 - m, axis=0, keepdims=True)
    bn = jnp.where(bn == 0.0, 1.0, bn)
    al = jnp.sum(cas * m, axis=0, keepdims=True) / an
    bl = jnp.sum(cas * (1.0 - m), axis=0, keepdims=True) / bn
    act_ref[0] = _masked_softmax(al, valid)
    bkg_ref[0] = _masked_softmax(bl, valid)


def _run_final(cas_sum, mask_pad):
    return pl.pallas_call(
        _final_body,
        grid=(_N,),
        in_specs=[
            pl.BlockSpec((1, _T, _OC), lambda n: (n, 0, 0)),
            pl.BlockSpec((1, _T, _OC), lambda n: (n, 0, 0)),
        ],
        out_specs=[
            pl.BlockSpec((1, 1, _OC), lambda n: (n, 0, 0)),
            pl.BlockSpec((1, 1, _OC), lambda n: (n, 0, 0)),
        ],
        out_shape=[
            jax.ShapeDtypeStruct((_N, 1, _OC), jnp.float32),
            jax.ShapeDtypeStruct((_N, 1, _OC), jnp.float32),
        ],
    )(cas_sum, mask_pad)


def kernel(x, cas_rgb_w1, cas_rgb_b1, cas_rgb_w2, cas_rgb_b2, cas_flow_w1,
           cas_flow_b1, cas_flow_w2, cas_flow_b2, aas_rgb_w1, aas_rgb_b1,
           aas_rgb_w2, aas_rgb_b2, aas_flow_w1, aas_flow_b1, aas_flow_w2,
           aas_flow_b2):
    xpad = jnp.pad(
        jnp.transpose(x.reshape(_N, _T, 2, 1024), (0, 2, 1, 3)),
        ((0, 0), (0, 0), (1, 1), (0, 0)))

    def _w1(w):
        return jnp.transpose(w, (2, 1, 0))

    def _w2(w):
        wt = jnp.transpose(w[:, :, 0], (1, 0))
        return jnp.pad(wt, ((0, 0), (0, _OC - wt.shape[1])))

    w1s = jnp.stack([_w1(cas_rgb_w1), _w1(cas_flow_w1),
                     _w1(aas_rgb_w1), _w1(aas_flow_w1)])
    b1s = jnp.stack([cas_rgb_b1, cas_flow_b1, aas_rgb_b1,
                     aas_flow_b1]).reshape(4, 1, 1024)
    w2s = jnp.stack([_w2(cas_rgb_w2), _w2(cas_flow_w2),
                     _w2(aas_rgb_w2), _w2(aas_flow_w2)])

    def _b2(b):
        return jnp.pad(b, (0, _OC - b.shape[0]))

    b2s = jnp.stack([_b2(cas_rgb_b2), _b2(cas_flow_b2),
                     _b2(aas_rgb_b2), _b2(aas_flow_b2)]).reshape(4, 1, _OC)

    out_all = _run_encoders(xpad, w1s, b1s, w2s, b2s)
    seg_pad, cas_sum, aas_sig = _run_combine(out_all)

    seg_score = seg_pad[:, :, :_C]
    seg_flat = jnp.transpose(seg_score, (0, 2, 1)).reshape(_R, _T)
    seg_rows = seg_flat.reshape(_R, 1, _T)
    seg_cols = seg_flat.reshape(_R, _T, 1)

    sorted_rt, rank_rt = _run_rank(seg_rows, seg_cols)
    sorted_cols = jnp.transpose(sorted_rt[:, :, 0], (1, 0))
    refined_cols = _run_cluster(sorted_cols)
    refined_rows = jnp.transpose(refined_cols, (1, 0)).reshape(_R, 1, _T)
    mask_rt = _run_unsort(rank_rt, refined_rows)

    seg_mask = jnp.transpose(mask_rt.reshape(_N, _C, _T), (0, 2, 1))
    mask_pad = jnp.pad(seg_mask, ((0, 0), (0, 0), (0, _OC - _C)))
    act_pad, bkg_pad = _run_final(cas_sum, mask_pad)

    act_score = act_pad[:, 0, :_C]
    bkg_score = bkg_pad[:, 0, :_C]
    aas = aas_sig[:, :, 0:1]
    return (act_score, bkg_score, aas, seg_score, seg_mask)


# speedup vs baseline: 6.6767x; 1.0010x over previous
import jax
import jax.numpy as jnp
from jax import lax
from jax.experimental import pallas as pl

_N, _T, _C = 4, 512, 20
_R = _N * _C
_OC = 32


def _enc_body(x_ref, w_ref, b1_ref, w2_ref, b2_ref, out_ref):
    xp = x_ref[0, 0, 0:_T, :]
    xc = x_ref[0, 0, 1:_T + 1, :]
    xn = x_ref[0, 0, 2:_T + 2, :]
    h = jnp.dot(xp, w_ref[0, 0], preferred_element_type=jnp.float32)
    h = h + jnp.dot(xc, w_ref[0, 1], preferred_element_type=jnp.float32)
    h = h + jnp.dot(xn, w_ref[0, 2], preferred_element_type=jnp.float32)
    h = jnp.maximum(h + b1_ref[0], 0.0)
    out = jnp.dot(h, w2_ref[0], preferred_element_type=jnp.float32)
    out_ref[0, 0] = out + b2_ref[0]


def _run_encoders(xpad, w1s, b1s, w2s, b2s):
    return pl.pallas_call(
        _enc_body,
        grid=(4, _N),
        in_specs=[
            pl.BlockSpec((1, 1, _T + 2, 1024), lambda e, n: (n, e % 2, 0, 0)),
            pl.BlockSpec((1, 3, 1024, 1024), lambda e, n: (e, 0, 0, 0)),
            pl.BlockSpec((1, 1, 1024), lambda e, n: (e, 0, 0)),
            pl.BlockSpec((1, 1024, _OC), lambda e, n: (e, 0, 0)),
            pl.BlockSpec((1, 1, _OC), lambda e, n: (e, 0, 0)),
        ],
        out_specs=pl.BlockSpec((1, 1, _T, _OC), lambda e, n: (e, n, 0, 0)),
        out_shape=jax.ShapeDtypeStruct((4, _N, _T, _OC), jnp.float32),
    )(xpad, w1s, b1s, w2s, b2s)


def _combine_body(o_ref, seg_ref, cassum_ref, aas_ref):
    cas = o_ref[0, 0] + o_ref[1, 0]
    aassum = o_ref[2, 0] + o_ref[3, 0]
    lanes = lax.broadcasted_iota(jnp.int32, (_T, _OC), 1)
    valid = lanes < _C
    mx = jnp.max(jnp.where(valid, cas, -jnp.inf), axis=1, keepdims=True)
    e = jnp.where(valid, jnp.exp(cas - mx), 0.0)
    soft = e / jnp.sum(e, axis=1, keepdims=True)
    sig = 1.0 / (1.0 + jnp.exp(-aassum))
    seg = (soft + sig[:, 0:1]) * 0.5
    seg_ref[0] = jnp.where(valid, seg, 0.0)
    cassum_ref[0] = jnp.where(valid, cas, 0.0)
    aas_ref[0] = sig


def _run_combine(out_all):
    return pl.pallas_call(
        _combine_body,
        grid=(_N,),
        in_specs=[pl.BlockSpec((4, 1, _T, _OC), lambda n: (0, n, 0, 0))],
        out_specs=[
            pl.BlockSpec((1, _T, _OC), lambda n: (n, 0, 0)),
            pl.BlockSpec((1, _T, _OC), lambda n: (n, 0, 0)),
            pl.BlockSpec((1, _T, _OC), lambda n: (n, 0, 0)),
        ],
        out_shape=[
            jax.ShapeDtypeStruct((_N, _T, _OC), jnp.float32),
            jax.ShapeDtypeStruct((_N, _T, _OC), jnp.float32),
            jax.ShapeDtypeStruct((_N, _T, _OC), jnp.float32),
        ],
    )(out_all)


def _rank_body(row_ref, col_ref, sorted_ref, rank_ref):
    b = row_ref[0]
    a = col_ref[0]
    iu = lax.broadcasted_iota(jnp.int32, (_T, _T), 0)
    it = lax.broadcasted_iota(jnp.int32, (_T, _T), 1)
    beats = ((a > b) | ((a == b) & (iu < it))).astype(jnp.float32)
    rank_lane = jnp.sum(beats, axis=0, keepdims=True)
    rank_sub = float(_T - 1) - jnp.sum(beats, axis=1, keepdims=True)
    isub = lax.broadcasted_iota(jnp.int32, (_T, _T), 0).astype(jnp.float32)
    onehot = (rank_lane == isub).astype(jnp.float32)
    sorted_ref[0] = jnp.sum(onehot * b, axis=1, keepdims=True)
    rank_ref[0] = rank_sub


def _run_rank(seg_rows, seg_cols):
    return pl.pallas_call(
        _rank_body,
        grid=(_R,),
        in_specs=[
            pl.BlockSpec((1, 1, _T), lambda r: (r, 0, 0)),
            pl.BlockSpec((1, _T, 1), lambda r: (r, 0, 0)),
        ],
        out_specs=[
            pl.BlockSpec((1, _T, 1), lambda r: (r, 0, 0)),
            pl.BlockSpec((1, _T, 1), lambda r: (r, 0, 0)),
        ],
        out_shape=[
            jax.ShapeDtypeStruct((_R, _T, 1), jnp.float32),
            jax.ShapeDtypeStruct((_R, _T, 1), jnp.float32),
        ],
    )(seg_rows, seg_cols)


def _cluster_body(sv_ref, out_ref):
    v0 = sv_ref[0:1, :]
    vlast = sv_ref[_T - 1:_T, :]
    out_ref[0:1, :] = jnp.ones_like(v0)
    out_ref[_T - 1:_T, :] = jnp.zeros_like(v0)

    def body(i, carry):
        ps, pn, ns_, nn = carry
        vi = sv_ref[pl.ds(i, 1), :]
        cond = (jnp.abs(vi - ps / pn) <= jnp.abs(vi - ns_ / nn)).astype(
            jnp.float32)
        out_ref[pl.ds(i, 1), :] = cond
        fi = (i + 1).astype(jnp.float32)
        return (ps + cond * vi / fi, pn + cond / fi,
                ns_ + (1.0 - cond) * vi, nn + (1.0 - cond))

    lax.fori_loop(1, _T - 1, body,
                  (v0, jnp.ones_like(v0), vlast, jnp.ones_like(v0)))
    mask = out_ref[...]
    sv = sv_ref[...]
    cnt = jnp.sum(mask, axis=0, keepdims=True)
    cnt = jnp.where(cnt == 0.0, 1.0, cnt)
    act = jnp.sum(mask * sv, axis=0, keepdims=True) / cnt
    out_ref[...] = jnp.where(sv >= act, mask, 0.0)


def _run_cluster(sorted_cols):
    return pl.pallas_call(
        _cluster_body,
        out_shape=jax.ShapeDtypeStruct((_T, _R), jnp.float32),
    )(sorted_cols)


def _unsort_body(rank_ref, refined_ref, out_ref):
    r = rank_ref[0]
    q = refined_ref[0]
    ilane = lax.broadcasted_iota(jnp.int32, (_T, _T), 1).astype(jnp.float32)
    sel = (r == ilane).astype(jnp.float32)
    out_ref[0] = jnp.sum(sel * q, axis=1, keepdims=True)


def _run_unsort(rank_rows, refined_rows):
    return pl.pallas_call(
        _unsort_body,
        grid=(_R,),
        in_specs=[
            pl.BlockSpec((1, _T, 1), lambda r: (r, 0, 0)),
            pl.BlockSpec((1, 1, _T), lambda r: (r, 0, 0)),
        ],
        out_specs=pl.BlockSpec((1, _T, 1), lambda r: (r, 0, 0)),
        out_shape=jax.ShapeDtypeStruct((_R, _T, 1), jnp.float32),
    )(rank_rows, refined_rows)


def _masked_softmax(x, valid):
    mx = jnp.max(jnp.where(valid, x, -jnp.inf), axis=1, keepdims=True)
    e = jnp.where(valid, jnp.exp(x - mx), 0.0)
    return e / jnp.sum(e, axis=1, keepdims=True)


def _final_body(cas_ref, mask_ref, act_ref, bkg_ref):
    cas = cas_ref[0][:, 0:_C]
    m = mask_ref[0]
    valid = lax.broadcasted_iota(jnp.int32, (1, _C), 1) < _C
    an = jnp.sum(m, axis=0, keepdims=True)
    an = jnp.where(an == 0.0, 1.0, an)
    bn = jnp.sum(1.0 - m, axis=0, keepdims=True)
    bn = jnp.where(bn == 0.0, 1.0, bn)
    al = jnp.sum(cas * m, axis=0, keepdims=True) / an
    bl = jnp.sum(cas * (1.0 - m), axis=0, keepdims=True) / bn
    act_ref[0] = _masked_softmax(al, valid)
    bkg_ref[0] = _masked_softmax(bl, valid)


def _run_final(cas_sum, seg_mask):
    return pl.pallas_call(
        _final_body,
        grid=(_N,),
        in_specs=[
            pl.BlockSpec((1, _T, _OC), lambda n: (n, 0, 0)),
            pl.BlockSpec((1, _T, _C), lambda n: (n, 0, 0)),
        ],
        out_specs=[
            pl.BlockSpec((1, 1, _C), lambda n: (n, 0, 0)),
            pl.BlockSpec((1, 1, _C), lambda n: (n, 0, 0)),
        ],
        out_shape=[
            jax.ShapeDtypeStruct((_N, 1, _C), jnp.float32),
            jax.ShapeDtypeStruct((_N, 1, _C), jnp.float32),
        ],
    )(cas_sum, seg_mask)


def kernel(x, cas_rgb_w1, cas_rgb_b1, cas_rgb_w2, cas_rgb_b2, cas_flow_w1,
           cas_flow_b1, cas_flow_w2, cas_flow_b2, aas_rgb_w1, aas_rgb_b1,
           aas_rgb_w2, aas_rgb_b2, aas_flow_w1, aas_flow_b1, aas_flow_w2,
           aas_flow_b2):
    xpad = jnp.pad(
        jnp.transpose(x.reshape(_N, _T, 2, 1024), (0, 2, 1, 3)),
        ((0, 0), (0, 0), (1, 1), (0, 0)))

    def _w1(w):
        return jnp.transpose(w, (2, 1, 0))

    def _w2(w):
        wt = jnp.transpose(w[:, :, 0], (1, 0))
        return jnp.pad(wt, ((0, 0), (0, _OC - wt.shape[1])))

    w1s = jnp.stack([_w1(cas_rgb_w1), _w1(cas_flow_w1),
                     _w1(aas_rgb_w1), _w1(aas_flow_w1)])
    b1s = jnp.stack([cas_rgb_b1, cas_flow_b1, aas_rgb_b1,
                     aas_flow_b1]).reshape(4, 1, 1024)
    w2s = jnp.stack([_w2(cas_rgb_w2), _w2(cas_flow_w2),
                     _w2(aas_rgb_w2), _w2(aas_flow_w2)])

    def _b2(b):
        return jnp.pad(b, (0, _OC - b.shape[0]))

    b2s = jnp.stack([_b2(cas_rgb_b2), _b2(cas_flow_b2),
                     _b2(aas_rgb_b2), _b2(aas_flow_b2)]).reshape(4, 1, _OC)

    out_all = _run_encoders(xpad, w1s, b1s, w2s, b2s)
    seg_pad, cas_sum, aas_sig = _run_combine(out_all)

    seg_score = seg_pad[:, :, :_C]
    seg_flat = jnp.transpose(seg_score, (0, 2, 1)).reshape(_R, _T)
    seg_rows = seg_flat.reshape(_R, 1, _T)
    seg_cols = seg_flat.reshape(_R, _T, 1)

    sorted_rt, rank_rt = _run_rank(seg_rows, seg_cols)
    sorted_cols = jnp.transpose(sorted_rt[:, :, 0], (1, 0))
    refined_cols = _run_cluster(sorted_cols)
    refined_rows = jnp.transpose(refined_cols, (1, 0)).reshape(_R, 1, _T)
    mask_rt = _run_unsort(rank_rt, refined_rows)

    seg_mask = jnp.transpose(mask_rt.reshape(_N, _C, _T), (0, 2, 1))
    act_pad, bkg_pad = _run_final(cas_sum, seg_mask)

    act_score = act_pad[:, 0, :]
    bkg_score = bkg_pad[:, 0, :]
    aas = aas_sig[:, :, 0:1]
    return (act_score, bkg_score, aas, seg_score, seg_mask)
